# baseline (speedup 1.0000x reference)
import functools

import jax
import jax.numpy as jnp
from jax import lax
from jax.experimental import pallas as pl
from jax.experimental.pallas import tpu as pltpu
from jax.experimental.pallas import tpu_sc as plsc

_N = 1_000_000
_BLK = 114_688
_GRID = -(-_N // _BLK)


def _copy_body(*refs):
    n = len(refs) // 2
    for i in range(n):
        refs[n + i][...] = refs[i][...]


def _tc_copy(args):
    specs = [pl.BlockSpec((a.shape[0], _BLK), lambda i: (0, i)) for a in args]
    out_shape = tuple(jax.ShapeDtypeStruct(a.shape, a.dtype) for a in args)
    return pl.pallas_call(
        _copy_body,
        grid=(_GRID,),
        out_shape=out_shape,
        in_specs=specs,
        out_specs=specs,
    )(*args)


def _sc_copy(z, vox, filt):
    mesh = plsc.ScalarSubcoreMesh(axis_name="core", num_cores=2)

    @functools.partial(
        pl.kernel,
        out_type=(jax.ShapeDtypeStruct(z.shape, z.dtype),
                  jax.ShapeDtypeStruct(vox.shape, vox.dtype),
                  jax.ShapeDtypeStruct(filt.shape, filt.dtype)),
        mesh=mesh,
        scratch_types=[pltpu.SemaphoreType.DMA((2,))],
    )
    def k(z_ref, v_ref, f_ref, zo, vo, fo, sems):
        cid = lax.axis_index("core")

        @pl.when(cid == 0)
        def _():
            c0 = pltpu.make_async_copy(z_ref, zo, sems.at[0])
            c1 = pltpu.make_async_copy(f_ref, fo, sems.at[1])
            c0.start()
            c1.start()
            c0.wait()
            c1.wait()

        @pl.when(cid == 1)
        def _():
            c = pltpu.make_async_copy(v_ref, vo, sems.at[0])
            c.start()
            c.wait()

    return k(z, vox, filt)


def kernel(new_xyz, new_colors, new_rots, new_scales, new_z_values,
           new_trackable_filter, new_voxel_index):
    wide = (new_xyz.T, new_colors.T, new_rots.T, new_scales.T)
    xyz_t, colors_t, rots_t, scales_t = _tc_copy(wide)
    z_out, vox_out, filt_out = _sc_copy(new_z_values, new_voxel_index,
                                        new_trackable_filter)
    return (xyz_t.T, colors_t.T, rots_t.T, scales_t.T, z_out, filt_out,
            vox_out)


# speedup vs baseline: 6.3026x; 6.3026x over previous
import jax
import jax.numpy as jnp
from jax.experimental import pallas as pl
from jax.experimental.pallas import tpu as pltpu

_N = 1_000_000
_F = 250_000
_BLK = 126_976
_GRID = -(-_N // _BLK)
_FBLK = 31_744


def _copy_body(*refs):
    n = len(refs) // 2
    for i in range(n):
        refs[n + i][...] = refs[i][...]


def kernel(new_xyz, new_colors, new_rots, new_scales, new_z_values,
           new_trackable_filter, new_voxel_index):
    args = (new_xyz.T, new_colors.T, new_rots.T, new_scales.T,
            new_z_values, new_trackable_filter, new_voxel_index)

    def _spec(shape):
        if len(shape) == 2:
            return pl.BlockSpec((shape[0], _BLK), lambda i: (0, i))
        if shape[0] == _F:
            return pl.BlockSpec((_FBLK,), lambda i: (i,))
        return pl.BlockSpec((_BLK,), lambda i: (i,))

    specs = [_spec(a.shape) for a in args]
    out_shape = tuple(jax.ShapeDtypeStruct(a.shape, a.dtype) for a in args)
    outs = pl.pallas_call(
        _copy_body,
        grid=(_GRID,),
        out_shape=out_shape,
        in_specs=specs,
        out_specs=specs,
    )(*args)
    return (outs[0].T, outs[1].T, outs[2].T, outs[3].T, outs[4], outs[5],
            outs[6])


# speedup vs baseline: 6.3981x; 1.0152x over previous
import jax
import jax.numpy as jnp
from jax.experimental import pallas as pl
from jax.experimental.pallas import tpu as pltpu

_N = 1_000_000
_F = 250_000
_BLK = 118_784
_GRID = -(-_N // _BLK)
_FBLK = 29_696


def _copy_body(*refs):
    n = len(refs) // 2
    for i in range(n):
        refs[n + i][...] = refs[i][...]


def kernel(new_xyz, new_colors, new_rots, new_scales, new_z_values,
           new_trackable_filter, new_voxel_index):
    args = (new_xyz.T, new_colors.T, new_rots.T, new_scales.T,
            new_z_values, new_trackable_filter, new_voxel_index)

    def _spec(shape):
        if len(shape) == 2:
            return pl.BlockSpec((shape[0], _BLK), lambda i: (0, i))
        if shape[0] == _F:
            return pl.BlockSpec((_FBLK,), lambda i: (i,))
        return pl.BlockSpec((_BLK,), lambda i: (i,))

    specs = [_spec(a.shape) for a in args]
    out_shape = tuple(jax.ShapeDtypeStruct(a.shape, a.dtype) for a in args)
    outs = pl.pallas_call(
        _copy_body,
        grid=(_GRID,),
        out_shape=out_shape,
        in_specs=specs,
        out_specs=specs,
    )(*args)
    return (outs[0].T, outs[1].T, outs[2].T, outs[3].T, outs[4], outs[5],
            outs[6])
